# baseline (speedup 1.0000x reference)
import functools

import jax
import jax.numpy as jnp
from jax import lax
from jax.experimental import pallas as pl
from jax.experimental.pallas import tpu as pltpu
from jax.experimental.pallas import tpu_sc as plsc

VOCAB = 50265
HIDDEN = 1024
PAD_IDX = 1
MAX_POS = 4098
LN_EPS = 1e-12
B, S = 4, 2048
TOK = B * S

NC = 2
NS = 16
NW = NC * NS
L = 16

BPW = TOK // NW
CH = 32
RCH = HIDDEN // L

_sc_mesh = plsc.VectorSubcoreMesh(core_axis_name="c", subcore_axis_name="s")


@functools.partial(
    pl.kernel,
    out_type=[
        jax.ShapeDtypeStruct((TOK, HIDDEN), jnp.float32),
        jax.ShapeDtypeStruct((TOK,), jnp.int32),
    ],
    mesh=_sc_mesh,
    scratch_types=[
        pltpu.VMEM((S,), jnp.int32),
        pltpu.VMEM((BPW,), jnp.int32),
        pltpu.VMEM((CH, HIDDEN), jnp.float32),
        pltpu.VMEM((CH, HIDDEN), jnp.float32),
        pltpu.SemaphoreType.DMA,
        pltpu.SemaphoreType.DMA,
    ],
)
def _sc_embed(ids_hbm, word_hbm, pos_hbm, out_hbm, pid_hbm,
              ids_v, pid_v, wbuf, pbuf, wsem, psem):
    wid = lax.axis_index("s") * NC + lax.axis_index("c")
    base = wid * BPW
    row = base // S
    col = base % S

    pltpu.sync_copy(ids_hbm.at[pl.ds(row * S, S)], ids_v)

    lane = lax.iota(jnp.int32, L)
    last = jnp.full((L,), L - 1, jnp.int32)

    def lane_gather(x, idx):
        dn = lax.GatherDimensionNumbers(
            offset_dims=(), collapsed_slice_dims=(0,), start_index_map=(0,))
        return lax.gather(x, idx[:, None], dn, slice_sizes=(1,),
                          mode=lax.GatherScatterMode.PROMISE_IN_BOUNDS)

    def scan_body(k, carry):
        ids16 = ids_v[pl.ds(k * L, L)]
        m = jnp.where(ids16 != PAD_IDX, 1, 0).astype(jnp.int32)
        cs = m
        for sh in (1, 2, 4, 8):
            g = lane_gather(cs, jnp.maximum(lane - sh, 0))
            cs = cs + jnp.where(lane >= sh, g, 0)
        tot = cs + carry
        off = k * L

        @pl.when(jnp.logical_and(off >= col, off < col + BPW))
        def _():
            pid_v[pl.ds(off - col, L)] = tot * m + PAD_IDX

        return lane_gather(tot, last)

    lax.fori_loop(0, (col + BPW) // L, scan_body,
                  jnp.zeros((L,), jnp.int32))

    pltpu.sync_copy(pid_v, pid_hbm.at[pl.ds(base, BPW)])

    def gather_body(c, _):
        off = c * CH
        cw = pltpu.async_copy(
            word_hbm.at[ids_v.at[pl.ds(col + off, CH)]], wbuf, wsem)
        cp = pltpu.async_copy(
            pos_hbm.at[pid_v.at[pl.ds(off, CH)]], pbuf, psem)
        cw.wait()
        cp.wait()

        def row_body(j, _):
            def col_body(k, _):
                sl = pl.ds(k * L, L)
                wbuf[j, sl] = wbuf[j, sl] + pbuf[j, sl]
                return 0
            lax.fori_loop(0, RCH, col_body, 0, unroll=8)
            return 0

        lax.fori_loop(0, CH, row_body, 0)
        pltpu.sync_copy(wbuf, out_hbm.at[pl.ds(base + off, CH)])
        return 0

    lax.fori_loop(0, BPW // CH, gather_body, 0)


def _ln_body(x_ref, t_ref, g_ref, b_ref, o_ref):
    x = x_ref[...] + t_ref[...]
    mu = jnp.mean(x, axis=-1, keepdims=True)
    xc = x - mu
    var = jnp.mean(xc * xc, axis=-1, keepdims=True)
    o_ref[...] = xc * lax.rsqrt(var + LN_EPS) * g_ref[...] + b_ref[...]


_LN_BLK = 512


def _layernorm_tc(summ, type_row, gamma, beta):
    grid = (TOK // _LN_BLK,)
    return pl.pallas_call(
        _ln_body,
        grid=grid,
        in_specs=[
            pl.BlockSpec((_LN_BLK, HIDDEN), lambda i: (i, 0)),
            pl.BlockSpec((1, HIDDEN), lambda i: (0, 0)),
            pl.BlockSpec((1, HIDDEN), lambda i: (0, 0)),
            pl.BlockSpec((1, HIDDEN), lambda i: (0, 0)),
        ],
        out_specs=pl.BlockSpec((_LN_BLK, HIDDEN), lambda i: (i, 0)),
        out_shape=jax.ShapeDtypeStruct((TOK, HIDDEN), jnp.float32),
    )(summ, type_row, gamma, beta)


def kernel(input_ids, word_emb, pos_emb, type_emb, ln_gamma, ln_beta):
    ids_flat = input_ids.reshape(TOK).astype(jnp.int32)
    summ, pid_flat = _sc_embed(ids_flat, word_emb, pos_emb)
    emb = _layernorm_tc(
        summ,
        type_emb[0:1],
        ln_gamma.reshape(1, HIDDEN),
        ln_beta.reshape(1, HIDDEN),
    )
    return (emb.reshape(B, S, HIDDEN),
            pid_flat.reshape(B, S).astype(input_ids.dtype))


# speedup vs baseline: 1.1546x; 1.1546x over previous
import functools

import jax
import jax.numpy as jnp
from jax import lax
from jax.experimental import pallas as pl
from jax.experimental.pallas import tpu as pltpu
from jax.experimental.pallas import tpu_sc as plsc

VOCAB = 50265
HIDDEN = 1024
PAD_IDX = 1
MAX_POS = 4098
LN_EPS = 1e-12
B, S = 4, 2048
TOK = B * S

NC = 2
NS = 16
NW = NC * NS
L = 16

BPW = TOK // NW
CH = 16
NCH = BPW // CH
CPB = 4
RCH = HIDDEN // L

_sc_mesh = plsc.VectorSubcoreMesh(core_axis_name="c", subcore_axis_name="s")


@functools.partial(
    pl.kernel,
    out_type=[
        jax.ShapeDtypeStruct((TOK, HIDDEN), jnp.float32),
        jax.ShapeDtypeStruct((TOK,), jnp.int32),
    ],
    mesh=_sc_mesh,
    scratch_types=(
        [pltpu.VMEM((S,), jnp.int32),
         pltpu.VMEM((BPW,), jnp.int32)]
        + [pltpu.VMEM((CH, HIDDEN), jnp.float32)] * 4
        + [pltpu.VMEM((CH, HIDDEN), jnp.float32)] * 3
        + [pltpu.SemaphoreType.DMA] * 11
    ),
)
def _sc_embed(ids_hbm, word_hbm, pos_hbm, out_hbm, pid_hbm,
              ids_v, pid_v, w0, w1, w2, w3, p0, p1, p2, *sems):
    wid = lax.axis_index("s") * NC + lax.axis_index("c")
    base = wid * BPW
    row = base // S
    col = base % S

    pltpu.sync_copy(ids_hbm.at[pl.ds(row * S, S)], ids_v)

    lane = lax.iota(jnp.int32, L)
    last = jnp.full((L,), L - 1, jnp.int32)

    def lane_gather(x, idx):
        dn = lax.GatherDimensionNumbers(
            offset_dims=(), collapsed_slice_dims=(0,), start_index_map=(0,))
        return lax.gather(x, idx[:, None], dn, slice_sizes=(1,),
                          mode=lax.GatherScatterMode.PROMISE_IN_BOUNDS)

    def scan_body(k, carry):
        ids16 = ids_v[pl.ds(k * L, L)]
        m = jnp.where(ids16 != PAD_IDX, 1, 0).astype(jnp.int32)
        cs = m
        for sh in (1, 2, 4, 8):
            g = lane_gather(cs, jnp.maximum(lane - sh, 0))
            cs = cs + jnp.where(lane >= sh, g, 0)
        tot = cs + carry
        off = k * L

        @pl.when(jnp.logical_and(off >= col, off < col + BPW))
        def _():
            pid_v[pl.ds(off - col, L)] = tot * m + PAD_IDX

        return lane_gather(tot, last)

    lax.fori_loop(0, (col + BPW) // L, scan_body,
                  jnp.zeros((L,), jnp.int32))

    pltpu.sync_copy(pid_v, pid_hbm.at[pl.ds(base, BPW)])

    wbufs = (w0, w1, w2, w3)
    pbufs = (p0, p1, p2)
    wsems = sems[0:4]
    psems = sems[4:7]
    osems = sems[7:11]

    def wgather(c, u):
        return pltpu.async_copy(
            word_hbm.at[ids_v.at[pl.ds(col + c * CH, CH)]],
            wbufs[u], wsems[u])

    def pgather(c, q):
        return pltpu.async_copy(
            pos_hbm.at[pid_v.at[pl.ds(c * CH, CH)]], pbufs[q], psems[q])

    def add_chunk(wb, pb):
        def row_body(j, _):
            def col_body(k, _):
                sl = pl.ds(k * L, L)
                plsc.addupdate(wb.at[j, sl], pb[j, sl])
                return 0
            lax.fori_loop(0, RCH, col_body, 0, unroll=8)
            return 0
        lax.fori_loop(0, CH, row_body, 0)

    def body(t, _):
        c0 = t * CPB
        wg = [wgather(c0 + u, u) for u in range(CPB)]
        pg = [pgather(c0 + q, q) for q in range(3)]
        outs = []
        for u in range(CPB):
            wg[u].wait()
            pg[u].wait()
            add_chunk(wbufs[u], pbufs[u % 3])
            outs.append(pltpu.async_copy(
                wbufs[u], out_hbm.at[pl.ds(base + (c0 + u) * CH, CH)],
                osems[u]))
            if u == 0:
                pg.append(pgather(c0 + 3, 0))
        for o in outs:
            o.wait()
        return 0

    lax.fori_loop(0, NCH // CPB, body, 0)


def _ln_body(x_ref, t_ref, g_ref, b_ref, o_ref):
    x = x_ref[...] + t_ref[...]
    mu = jnp.mean(x, axis=-1, keepdims=True)
    xc = x - mu
    var = jnp.mean(xc * xc, axis=-1, keepdims=True)
    o_ref[...] = xc * lax.rsqrt(var + LN_EPS) * g_ref[...] + b_ref[...]


_LN_BLK = 512


def _layernorm_tc(summ, type_row, gamma, beta):
    grid = (TOK // _LN_BLK,)
    return pl.pallas_call(
        _ln_body,
        grid=grid,
        in_specs=[
            pl.BlockSpec((_LN_BLK, HIDDEN), lambda i: (i, 0)),
            pl.BlockSpec((1, HIDDEN), lambda i: (0, 0)),
            pl.BlockSpec((1, HIDDEN), lambda i: (0, 0)),
            pl.BlockSpec((1, HIDDEN), lambda i: (0, 0)),
        ],
        out_specs=pl.BlockSpec((_LN_BLK, HIDDEN), lambda i: (i, 0)),
        out_shape=jax.ShapeDtypeStruct((TOK, HIDDEN), jnp.float32),
    )(summ, type_row, gamma, beta)


def kernel(input_ids, word_emb, pos_emb, type_emb, ln_gamma, ln_beta):
    ids_flat = input_ids.reshape(TOK).astype(jnp.int32)
    summ, pid_flat = _sc_embed(ids_flat, word_emb, pos_emb)
    emb = _layernorm_tc(
        summ,
        type_emb[0:1],
        ln_gamma.reshape(1, HIDDEN),
        ln_beta.reshape(1, HIDDEN),
    )
    return (emb.reshape(B, S, HIDDEN),
            pid_flat.reshape(B, S).astype(input_ids.dtype))


# speedup vs baseline: 1.6316x; 1.4132x over previous
import functools

import jax
import jax.numpy as jnp
from jax import lax
from jax.experimental import pallas as pl
from jax.experimental.pallas import tpu as pltpu
from jax.experimental.pallas import tpu_sc as plsc

VOCAB = 50265
HIDDEN = 1024
PAD_IDX = 1
MAX_POS = 4098
LN_EPS = 1e-12
B, S = 4, 2048
TOK = B * S

NC = 2
NS = 16
NW = NC * NS
L = 16

BPW = TOK // NW
CH = 16
NCH = BPW // CH
CPB = 4
RCH = HIDDEN // L

_sc_mesh = plsc.VectorSubcoreMesh(core_axis_name="c", subcore_axis_name="s")


@functools.partial(
    pl.kernel,
    out_type=[
        jax.ShapeDtypeStruct((TOK, HIDDEN), jnp.float32),
        jax.ShapeDtypeStruct((TOK,), jnp.int32),
    ],
    mesh=_sc_mesh,
    scratch_types=(
        [pltpu.VMEM((S,), jnp.int32),
         pltpu.VMEM((BPW,), jnp.int32)]
        + [pltpu.VMEM((CH, HIDDEN), jnp.float32)] * 4
        + [pltpu.VMEM((CH, HIDDEN), jnp.float32)] * 3
        + [pltpu.SemaphoreType.DMA] * 11
    ),
)
def _sc_embed(ids_hbm, word_hbm, pos_hbm, out_hbm, pid_hbm,
              ids_v, pid_v, w0, w1, w2, w3, p0, p1, p2, *sems):
    wid = lax.axis_index("s") * NC + lax.axis_index("c")
    base = wid * BPW
    row = base // S
    col = base % S

    pltpu.sync_copy(ids_hbm.at[pl.ds(row * S, S)], ids_v)

    lane = lax.iota(jnp.int32, L)
    last = jnp.full((L,), L - 1, jnp.int32)

    def lane_gather(x, idx):
        dn = lax.GatherDimensionNumbers(
            offset_dims=(), collapsed_slice_dims=(0,), start_index_map=(0,))
        return lax.gather(x, idx[:, None], dn, slice_sizes=(1,),
                          mode=lax.GatherScatterMode.PROMISE_IN_BOUNDS)

    def scan_body(k, carry):
        ids16 = ids_v[pl.ds(k * L, L)]
        m = jnp.where(ids16 != PAD_IDX, 1, 0).astype(jnp.int32)
        cs = m
        for sh in (1, 2, 4, 8):
            g = lane_gather(cs, jnp.maximum(lane - sh, 0))
            cs = cs + jnp.where(lane >= sh, g, 0)
        tot = cs + carry
        off = k * L

        @pl.when(jnp.logical_and(off >= col, off < col + BPW))
        def _():
            pid_v[pl.ds(off - col, L)] = tot * m + PAD_IDX

        return lane_gather(tot, last)

    lax.fori_loop(0, (col + BPW) // L, scan_body,
                  jnp.zeros((L,), jnp.int32))

    pltpu.sync_copy(pid_v, pid_hbm.at[pl.ds(base, BPW)])

    wbufs = (w0, w1, w2, w3)
    pbufs = (p0, p1, p2)
    wsems = sems[0:4]
    psems = sems[4:7]
    osems = sems[7:11]

    def wgather(c, u):
        return pltpu.async_copy(
            word_hbm.at[ids_v.at[pl.ds(col + c * CH, CH)]],
            wbufs[u], wsems[u])

    def pgather(c, q):
        return pltpu.async_copy(
            pos_hbm.at[pid_v.at[pl.ds(c * CH, CH)]], pbufs[q], psems[q])

    def add_chunk(wb, pb):
        def row_body(j, _):
            for k in range(RCH):
                sl = pl.ds(k * L, L)
                plsc.addupdate(wb.at[j, sl], pb[j, sl])
            return 0
        lax.fori_loop(0, CH, row_body, 0)

    def body(t, _):
        c0 = t * CPB
        wg = [wgather(c0 + u, u) for u in range(CPB)]
        pg = [pgather(c0 + q, q) for q in range(3)]
        outs = []
        for u in range(CPB):
            wg[u].wait()
            pg[u].wait()
            add_chunk(wbufs[u], pbufs[u % 3])
            outs.append(pltpu.async_copy(
                wbufs[u], out_hbm.at[pl.ds(base + (c0 + u) * CH, CH)],
                osems[u]))
            if u == 0:
                pg.append(pgather(c0 + 3, 0))
        for o in outs:
            o.wait()
        return 0

    lax.fori_loop(0, NCH // CPB, body, 0)


def _ln_body(x_ref, t_ref, g_ref, b_ref, o_ref):
    x = x_ref[...] + t_ref[...]
    mu = jnp.mean(x, axis=-1, keepdims=True)
    xc = x - mu
    var = jnp.mean(xc * xc, axis=-1, keepdims=True)
    o_ref[...] = xc * lax.rsqrt(var + LN_EPS) * g_ref[...] + b_ref[...]


_LN_BLK = 512


def _layernorm_tc(summ, type_row, gamma, beta):
    grid = (TOK // _LN_BLK,)
    return pl.pallas_call(
        _ln_body,
        grid=grid,
        in_specs=[
            pl.BlockSpec((_LN_BLK, HIDDEN), lambda i: (i, 0)),
            pl.BlockSpec((1, HIDDEN), lambda i: (0, 0)),
            pl.BlockSpec((1, HIDDEN), lambda i: (0, 0)),
            pl.BlockSpec((1, HIDDEN), lambda i: (0, 0)),
        ],
        out_specs=pl.BlockSpec((_LN_BLK, HIDDEN), lambda i: (i, 0)),
        out_shape=jax.ShapeDtypeStruct((TOK, HIDDEN), jnp.float32),
    )(summ, type_row, gamma, beta)


def kernel(input_ids, word_emb, pos_emb, type_emb, ln_gamma, ln_beta):
    ids_flat = input_ids.reshape(TOK).astype(jnp.int32)
    summ, pid_flat = _sc_embed(ids_flat, word_emb, pos_emb)
    emb = _layernorm_tc(
        summ,
        type_emb[0:1],
        ln_gamma.reshape(1, HIDDEN),
        ln_beta.reshape(1, HIDDEN),
    )
    return (emb.reshape(B, S, HIDDEN),
            pid_flat.reshape(B, S).astype(input_ids.dtype))
